# baseline (speedup 1.0000x reference)
import functools

import jax
import jax.numpy as jnp
from jax.experimental import pallas as pl
from jax.experimental.pallas import tpu as pltpu


def _head_kernel(x_ref, l_ref, u_ref, b_ref, o_ref, *, cin, th, width, rowdup):
    w2 = 2 * width
    rr = 16 if rowdup else 8
    for r in range(0, th, 8):
        m = x_ref[:, r:r + 8, :].reshape(cin * 8, width).astype(jnp.bfloat16)
        acc = jnp.dot(l_ref[...], m,
                      preferred_element_type=jnp.float32)
        up = jnp.dot(acc.astype(jnp.bfloat16), u_ref[...],
                     preferred_element_type=jnp.float32)
        up = (up + b_ref[...]).reshape(7, rr, w2)
        if rowdup:
            for o in range(7):
                o_ref[o, 2 * r:2 * r + 16, :] = up[o]
        else:
            for o in range(7):
                for t in range(8):
                    o_ref[o, 2 * (r + t):2 * (r + t) + 2, :] = jnp.broadcast_to(
                        up[o, t:t + 1, :], (2, w2))


def _conv1x1_head(x, w, s, sh, th, rowdup):
    N, C, H, W = x.shape
    xm = x.reshape(N * C, H, W)
    w7 = (w * s[None, :]).T[:7].astype(jnp.float32)
    rr = 16 if rowdup else 8
    dup = jnp.repeat(jnp.eye(8, dtype=jnp.float32), 2, axis=0) if rowdup \
        else jnp.eye(8, dtype=jnp.float32)
    lmat = jnp.einsum('oc,hk->ohck', w7, dup)
    lmat = lmat.reshape(7 * rr, C * 8).astype(jnp.bfloat16)
    bcol = jnp.repeat(sh[:7].astype(jnp.float32), rr).reshape(7 * rr, 1)
    u = jnp.equal(jnp.arange(W)[:, None],
                  jnp.arange(2 * W)[None, :] // 2).astype(jnp.bfloat16)

    grid = (N, H // th)
    out_shape = jax.ShapeDtypeStruct((N * 7, 2 * H, 2 * W), jnp.float32)
    flops = 2 * N * (H // 8) * 56 * C * 8 * W + 2 * N * H * 8 * W * 2 * W
    bytes_accessed = int(x.size * 4 + lmat.size * 4 + N * 7 * 4 * H * W * 4)

    body = functools.partial(_head_kernel, cin=C, th=th, width=W, rowdup=rowdup)
    y = pl.pallas_call(
        body,
        out_shape=out_shape,
        grid=grid,
        in_specs=[
            pl.BlockSpec((C, th, W), lambda n, t: (n, t, 0)),
            pl.BlockSpec((7 * rr, C * 8), lambda n, t: (0, 0)),
            pl.BlockSpec((W, 2 * W), lambda n, t: (0, 0)),
            pl.BlockSpec((7 * rr, 1), lambda n, t: (0, 0)),
        ],
        out_specs=pl.BlockSpec((7, 2 * th, 2 * W), lambda n, t: (n, t, 0)),
        compiler_params=pltpu.CompilerParams(
            dimension_semantics=("parallel", "parallel")),
        cost_estimate=pl.CostEstimate(flops=flops, transcendentals=0,
                                      bytes_accessed=bytes_accessed),
    )(xm, lmat, u, bcol)
    return y.reshape(N, 7, 2 * H, 2 * W)


def kernel(w_pre, s_pre, sh_pre, block0_w1x, block0_w1s, block0_s1, block0_sh1, block0_w2, block0_s2, block0_sh2, block0_wr, block0_sr, block0_shr, block1_w1x, block1_w1s, block1_s1, block1_sh1, block1_w2, block1_s2, block1_sh2, block1_wr, block1_sr, block1_shr, block2_w1x, block2_w1s, block2_s1, block2_sh1, block2_w2, block2_s2, block2_sh2, block2_wr, block2_sr, block2_shr, block3_w1x, block3_w1s, block3_s1, block3_sh1, block3_w2, block3_s2, block3_sh2, block3_wr, block3_sr, block3_shr, w_out_full, s_out_full, sh_out_full, w_out_half, s_out_half, sh_out_half, w_out_quarter, s_out_quarter, sh_out_quarter, x, x_prebottle, x_quarter, x_half, x_full):
    return (_conv1x1_head(x_full, w_out_full, s_out_full, sh_out_full, 16, True),
            _conv1x1_head(x_half, w_out_half, s_out_half, sh_out_half, 16, False),
            _conv1x1_head(x_quarter, w_out_quarter, s_out_quarter, sh_out_quarter, 16, False))


# speedup vs baseline: 1.4080x; 1.4080x over previous
import functools

import jax
import jax.numpy as jnp
from jax.experimental import pallas as pl
from jax.experimental.pallas import tpu as pltpu


def _head_kernel(x_ref, l_ref, u_ref, b_ref, o_ref, *, cin, th, width, rowdup):
    w2 = 2 * width
    rr = 16 if rowdup else 8
    for r in range(0, th, 8):
        m = x_ref[:, r:r + 8, :].reshape(cin * 8, width).astype(jnp.bfloat16)
        acc = jnp.dot(l_ref[...], m,
                      preferred_element_type=jnp.float32)
        up = jnp.dot(acc.astype(jnp.bfloat16), u_ref[...],
                     preferred_element_type=jnp.float32)
        up = (up + b_ref[...]).reshape(7, rr, w2)
        if rowdup:
            for o in range(7):
                o_ref[o, 2 * r:2 * r + 16, :] = up[o]
        else:
            for o in range(7):
                for t in range(8):
                    o_ref[o, 2 * (r + t):2 * (r + t) + 2, :] = jnp.broadcast_to(
                        up[o, t:t + 1, :], (2, w2))


def _conv1x1_head(x, w, s, sh, th, rowdup):
    N, C, H, W = x.shape
    xm = x.reshape(N * C, H, W)
    w7 = (w * s[None, :]).T[:7].astype(jnp.float32)
    rr = 16 if rowdup else 8
    dup = jnp.repeat(jnp.eye(8, dtype=jnp.float32), 2, axis=0) if rowdup \
        else jnp.eye(8, dtype=jnp.float32)
    lmat = jnp.einsum('oc,hk->ohck', w7, dup)
    lmat = lmat.reshape(7 * rr, C * 8).astype(jnp.bfloat16)
    bcol = jnp.repeat(sh[:7].astype(jnp.float32), rr).reshape(7 * rr, 1)
    u = jnp.equal(jnp.arange(W)[:, None],
                  jnp.arange(2 * W)[None, :] // 2).astype(jnp.bfloat16)

    grid = (N, H // th)
    out_shape = jax.ShapeDtypeStruct((N * 7, 2 * H, 2 * W), jnp.float32)
    flops = 2 * N * (H // 8) * 56 * C * 8 * W + 2 * N * H * 8 * W * 2 * W
    bytes_accessed = int(x.size * 4 + lmat.size * 4 + N * 7 * 4 * H * W * 4)

    body = functools.partial(_head_kernel, cin=C, th=th, width=W, rowdup=rowdup)
    y = pl.pallas_call(
        body,
        out_shape=out_shape,
        grid=grid,
        in_specs=[
            pl.BlockSpec((C, th, W), lambda n, t: (n, t, 0)),
            pl.BlockSpec((7 * rr, C * 8), lambda n, t: (0, 0)),
            pl.BlockSpec((W, 2 * W), lambda n, t: (0, 0)),
            pl.BlockSpec((7 * rr, 1), lambda n, t: (0, 0)),
        ],
        out_specs=pl.BlockSpec((7, 2 * th, 2 * W), lambda n, t: (n, t, 0)),
        compiler_params=pltpu.CompilerParams(
            dimension_semantics=("parallel", "parallel")),
        cost_estimate=pl.CostEstimate(flops=flops, transcendentals=0,
                                      bytes_accessed=bytes_accessed),
    )(xm, lmat, u, bcol)
    return y.reshape(N, 7, 2 * H, 2 * W)


def kernel(w_pre, s_pre, sh_pre, block0_w1x, block0_w1s, block0_s1, block0_sh1, block0_w2, block0_s2, block0_sh2, block0_wr, block0_sr, block0_shr, block1_w1x, block1_w1s, block1_s1, block1_sh1, block1_w2, block1_s2, block1_sh2, block1_wr, block1_sr, block1_shr, block2_w1x, block2_w1s, block2_s1, block2_sh1, block2_w2, block2_s2, block2_sh2, block2_wr, block2_sr, block2_shr, block3_w1x, block3_w1s, block3_s1, block3_sh1, block3_w2, block3_s2, block3_sh2, block3_wr, block3_sr, block3_shr, w_out_full, s_out_full, sh_out_full, w_out_half, s_out_half, sh_out_half, w_out_quarter, s_out_quarter, sh_out_quarter, x, x_prebottle, x_quarter, x_half, x_full):
    return (_conv1x1_head(x_full, w_out_full, s_out_full, sh_out_full, 64, True),
            _conv1x1_head(x_half, w_out_half, s_out_half, sh_out_half, 64, False),
            _conv1x1_head(x_quarter, w_out_quarter, s_out_quarter, sh_out_quarter, 32, False))


# speedup vs baseline: 1.4597x; 1.0367x over previous
import functools

import jax
import jax.numpy as jnp
from jax.experimental import pallas as pl
from jax.experimental.pallas import tpu as pltpu


def _head_kernel(x_ref, l_ref, u_ref, b_ref, o_ref, *, cin, th, width, rowdup):
    w2 = 2 * width
    rr = 16 if rowdup else 8
    for r in range(0, th, 8):
        m = x_ref[:, r:r + 8, :].reshape(cin * 8, width).astype(jnp.bfloat16)
        acc = jnp.dot(l_ref[...], m,
                      preferred_element_type=jnp.float32)
        up = jnp.dot(acc.astype(jnp.bfloat16), u_ref[...],
                     preferred_element_type=jnp.float32)
        up = (up + b_ref[...]).reshape(7, rr, w2)
        if rowdup:
            for o in range(7):
                o_ref[o, 2 * r:2 * r + 16, :] = up[o]
        else:
            for o in range(7):
                for t in range(8):
                    o_ref[o, 2 * (r + t):2 * (r + t) + 2, :] = jnp.broadcast_to(
                        up[o, t:t + 1, :], (2, w2))


def _conv1x1_head(x, w, s, sh, th, rowdup):
    N, C, H, W = x.shape
    xm = x.reshape(N * C, H, W)
    w7 = (w * s[None, :]).T[:7].astype(jnp.float32)
    rr = 16 if rowdup else 8
    dup = jnp.repeat(jnp.eye(8, dtype=jnp.float32), 2, axis=0) if rowdup \
        else jnp.eye(8, dtype=jnp.float32)
    lmat = jnp.einsum('oc,hk->ohck', w7, dup)
    lmat = lmat.reshape(7 * rr, C * 8).astype(jnp.bfloat16)
    bcol = jnp.repeat(sh[:7].astype(jnp.float32), rr).reshape(7 * rr, 1)
    u = jnp.equal(jnp.arange(W)[:, None],
                  jnp.arange(2 * W)[None, :] // 2).astype(jnp.bfloat16)

    grid = (N, H // th)
    out_shape = jax.ShapeDtypeStruct((N * 7, 2 * H, 2 * W), jnp.float32)
    flops = 2 * N * (H // 8) * 56 * C * 8 * W + 2 * N * H * 8 * W * 2 * W
    bytes_accessed = int(x.size * 4 + lmat.size * 4 + N * 7 * 4 * H * W * 4)

    body = functools.partial(_head_kernel, cin=C, th=th, width=W, rowdup=rowdup)
    y = pl.pallas_call(
        body,
        out_shape=out_shape,
        grid=grid,
        in_specs=[
            pl.BlockSpec((C, th, W), lambda n, t: (n, t, 0)),
            pl.BlockSpec((7 * rr, C * 8), lambda n, t: (0, 0)),
            pl.BlockSpec((W, 2 * W), lambda n, t: (0, 0)),
            pl.BlockSpec((7 * rr, 1), lambda n, t: (0, 0)),
        ],
        out_specs=pl.BlockSpec((7, 2 * th, 2 * W), lambda n, t: (n, t, 0)),
        compiler_params=pltpu.CompilerParams(
            dimension_semantics=("parallel", "parallel")),
        cost_estimate=pl.CostEstimate(flops=flops, transcendentals=0,
                                      bytes_accessed=bytes_accessed),
    )(xm, lmat, u, bcol)
    return y.reshape(N, 7, 2 * H, 2 * W)


def kernel(w_pre, s_pre, sh_pre, block0_w1x, block0_w1s, block0_s1, block0_sh1, block0_w2, block0_s2, block0_sh2, block0_wr, block0_sr, block0_shr, block1_w1x, block1_w1s, block1_s1, block1_sh1, block1_w2, block1_s2, block1_sh2, block1_wr, block1_sr, block1_shr, block2_w1x, block2_w1s, block2_s1, block2_sh1, block2_w2, block2_s2, block2_sh2, block2_wr, block2_sr, block2_shr, block3_w1x, block3_w1s, block3_s1, block3_sh1, block3_w2, block3_s2, block3_sh2, block3_wr, block3_sr, block3_shr, w_out_full, s_out_full, sh_out_full, w_out_half, s_out_half, sh_out_half, w_out_quarter, s_out_quarter, sh_out_quarter, x, x_prebottle, x_quarter, x_half, x_full):
    return (_conv1x1_head(x_full, w_out_full, s_out_full, sh_out_full, 128, True),
            _conv1x1_head(x_half, w_out_half, s_out_half, sh_out_half, 64, False),
            _conv1x1_head(x_quarter, w_out_quarter, s_out_quarter, sh_out_quarter, 32, False))
